# baseline (speedup 1.0000x reference)
import functools

import jax
import jax.numpy as jnp
from jax import lax
from jax.experimental import pallas as pl
from jax.experimental.pallas import tpu as pltpu
from jax.experimental.pallas import tpu_sc as plsc

NX, NY = 432, 496
NUM_BEV = 64
VOXEL_X, VOXEL_Y = 0.16, 0.16
X_RANGE, Y_RANGE = 0.0, -39.68
P, B, M = 40000, 4, 40
PP = 40960
ROWS = PP // 128
NXP = 512
SP = NY * NXP
BSP = B * SP
NROWS = 1056768
ZCH = 4128
BY = 16
JY = NY // BY


def _prep_body(bidx_ref, yc_ref, xc_ref, gt_ref, f_ref,
               dest_ref, code_ref, pf_ref, trig_ref):
    g = pl.program_id(0)

    @pl.when(g == 0)
    def _():
        h = gt_ref[6:7, :]
        trig_ref[0:1, :] = jnp.cos(h)
        trig_ref[1:2, :] = jnp.sin(h)

    pf_ref[:, 0:NUM_BEV] = f_ref[...]

    b = bidx_ref[...]
    y = yc_ref[...]
    x = xc_ref[...]
    px = x.astype(jnp.float32) * VOXEL_X + X_RANGE
    py = y.astype(jnp.float32) * VOXEL_Y + Y_RANGE
    cls = jnp.full((16, 128), -1, jnp.int32)
    for bb in range(B):
        bm = b == bb
        for m in range(M):
            k = bb * M + m
            cx = gt_ref[0, k]
            cy = gt_ref[1, k]
            dx2 = gt_ref[3, k] * 0.5
            dy2 = gt_ref[4, k] * 0.5
            ch = trig_ref[0, k]
            sh = trig_ref[1, k]
            ddx = px - cx
            ddy = py - cy
            lx = ddx * ch + ddy * sh
            ly = ddy * ch - ddx * sh
            hit = (jnp.abs(lx) <= dx2) & (jnp.abs(ly) <= dy2) & bm & (cls < 0)
            cls = jnp.where(hit, m, cls)
    r = lax.broadcasted_iota(jnp.int32, (16, 128), 0)
    c = lax.broadcasted_iota(jnp.int32, (16, 128), 1)
    p = (g * 16 + r) * 128 + c
    dest = b * SP + y * NXP + x
    dest_ref[...] = jnp.where(p < P, dest, BSP + p)
    code_ref[...] = p * 64 + (cls + 2)


_prep = pl.pallas_call(
    _prep_body,
    grid=(ROWS // 16,),
    in_specs=[
        pl.BlockSpec((16, 128), lambda g: (g, 0)),
        pl.BlockSpec((16, 128), lambda g: (g, 0)),
        pl.BlockSpec((16, 128), lambda g: (g, 0)),
        pl.BlockSpec((8, 256), lambda g: (0, 0)),
        pl.BlockSpec((2048, NUM_BEV), lambda g: (g, 0)),
    ],
    out_specs=[
        pl.BlockSpec((16, 128), lambda g: (g, 0)),
        pl.BlockSpec((16, 128), lambda g: (g, 0)),
        pl.BlockSpec((2048, 128), lambda g: (g, 0)),
    ],
    out_shape=[
        jax.ShapeDtypeStruct((ROWS, 128), jnp.int32),
        jax.ShapeDtypeStruct((ROWS, 128), jnp.int32),
        jax.ShapeDtypeStruct((PP, 128), jnp.float32),
    ],
    scratch_shapes=[pltpu.VMEM((8, 256), jnp.float32)],
)

_mesh = plsc.VectorSubcoreMesh(core_axis_name="c", subcore_axis_name="s")


@functools.partial(
    pl.kernel,
    mesh=_mesh,
    out_type=jax.ShapeDtypeStruct((NROWS,), jnp.int32),
    scratch_types=[
        pltpu.VMEM((ZCH,), jnp.int32),
        pltpu.VMEM((16, 128), jnp.int32),
        pltpu.VMEM((16, 128), jnp.int32),
        pltpu.SemaphoreType.DMA,
    ],
)
def _sc_map(dest_hbm, code_hbm, map_hbm, zbuf, d_v, c_v, sem):
    cid = lax.axis_index("c")
    sid = lax.axis_index("s")

    @pl.when(cid == 0)
    def _():
        @pl.loop(0, ZCH, step=16)
        def _(i):
            zbuf[pl.ds(i, 16)] = jnp.zeros((16,), jnp.int32)

        base = sid * (NROWS // 16)

        @pl.loop(0, NROWS // 16 // ZCH)
        def _(j):
            pltpu.sync_copy(zbuf, map_hbm.at[pl.ds(base + j * ZCH, ZCH)])

        plsc.subcore_barrier()

        @pl.when(sid == 0)
        def _():
            @pl.loop(0, ROWS // 16)
            def _(g):
                pltpu.sync_copy(dest_hbm.at[pl.ds(g * 16, 16)], d_v)
                pltpu.sync_copy(code_hbm.at[pl.ds(g * 16, 16)], c_v)
                for j in range(16):
                    pltpu.async_copy(c_v.at[j], map_hbm.at[d_v.at[j]], sem).wait()


@functools.partial(
    pl.kernel,
    mesh=_mesh,
    out_type=jax.ShapeDtypeStruct((NROWS, 128), jnp.float32),
    scratch_types=[
        pltpu.VMEM((128,), jnp.int32),
        pltpu.VMEM((128,), jnp.int32),
        pltpu.VMEM((128,), jnp.int32),
        pltpu.VMEM((128,), jnp.int32),
        pltpu.VMEM((128, 128), jnp.float32),
        pltpu.SemaphoreType.DMA,
    ],
)
def _sc_scatter(dest_hbm, code_hbm, map_hbm, feat_hbm, out_hbm,
                d_v, c_v, m_v, d2_v, f_v, sem):
    cid = lax.axis_index("c")
    sid = lax.axis_index("s")
    wid = sid * 2 + cid

    @pl.loop(0, ROWS // 32)
    def _(i):
        row = wid * (ROWS // 32) + i
        pltpu.sync_copy(dest_hbm.at[row], d_v)
        pltpu.sync_copy(code_hbm.at[row], c_v)
        pltpu.async_copy(map_hbm.at[d_v], m_v, sem).wait()
        pltpu.sync_copy(feat_hbm.at[pl.ds(row * 128, 128)], f_v)
        for k in range(8):
            sl = pl.ds(k * 16, 16)
            dd = d_v[sl]
            cc = c_v[sl]
            mm = m_v[sl]
            lane = lax.iota(jnp.int32, 16)
            dump = BSP + row * 128 + k * 16 + lane
            d2_v[sl] = jnp.where(mm == cc, dd, dump)
        pltpu.async_copy(f_v, out_hbm.at[d2_v], sem).wait()


def _fin_body(w_ref, b_ref, map_ref, feat_ref, sp_ref, off_ref, cls_ref):
    m2 = map_ref[...]
    valid2 = m2 > 0
    cls_ref[0] = jnp.where(valid2, (m2 & 63) - 2, -1)[:, :NX]
    for y in range(BY):
        t = feat_ref[pl.ds(y * NXP, NXP), 0:NUM_BEV].T
        v = valid2[y:y + 1, :]
        tm = jnp.where(v, t, 0.0)
        sp_ref[0, :, y, :] = tm[:, :NX]
        o = lax.dot_general(w_ref[...], tm, (((1,), (0,)), ((), ())),
                            preferred_element_type=jnp.float32)
        o = o + v.astype(jnp.float32) * b_ref[:, 0:1]
        off_ref[0, :, y, :] = o[0:2, :NX]


_fin = pl.pallas_call(
    _fin_body,
    grid=(B, JY),
    in_specs=[
        pl.BlockSpec((8, NUM_BEV), lambda b, j: (0, 0)),
        pl.BlockSpec((8, 8), lambda b, j: (0, 0)),
        pl.BlockSpec((BY, NXP), lambda b, j: (b * JY + j, 0)),
        pl.BlockSpec((BY * NXP, 128), lambda b, j: (b * JY + j, 0)),
    ],
    out_specs=[
        pl.BlockSpec((1, NUM_BEV, BY, NX), lambda b, j: (b, 0, j, 0)),
        pl.BlockSpec((1, 2, BY, NX), lambda b, j: (b, 0, j, 0)),
        pl.BlockSpec((1, BY, NX), lambda b, j: (b, j, 0)),
    ],
    out_shape=[
        jax.ShapeDtypeStruct((B, NUM_BEV, NY, NX), jnp.float32),
        jax.ShapeDtypeStruct((B, 2, NY, NX), jnp.float32),
        jax.ShapeDtypeStruct((B, NY, NX), jnp.int32),
    ],
)


def kernel(pillar_features, voxel_coords, gt_boxes, W_vote, b_vote):
    coords = jnp.pad(voxel_coords, ((0, PP - P), (0, 0)))
    bidx2 = coords[:, 0].reshape(ROWS, 128)
    yc2 = coords[:, 2].reshape(ROWS, 128)
    xc2 = coords[:, 3].reshape(ROWS, 128)
    gtt = jnp.pad(gt_boxes[..., :7].reshape(B * M, 7).T, ((0, 1), (0, 96)))
    w8 = jnp.zeros((8, NUM_BEV), jnp.float32).at[:2].set(W_vote)
    b8 = jnp.zeros((8, 8), jnp.float32).at[:2, 0].set(b_vote)

    dest2d, code2d, pf = _prep(bidx2, yc2, xc2, gtt, pillar_features)
    cmap = _sc_map(dest2d, code2d)
    feats = _sc_scatter(dest2d, code2d, cmap, pf)
    map2 = cmap.reshape(NROWS // NXP, NXP)
    spatial_features, offT, batch_cls = _fin(w8, b8, map2, feats)
    centering_offset = offT.transpose(0, 2, 3, 1)
    return spatial_features, centering_offset, batch_cls


# speedup vs baseline: 1.0004x; 1.0004x over previous
import functools

import jax
import jax.numpy as jnp
from jax import lax
from jax.experimental import pallas as pl
from jax.experimental.pallas import tpu as pltpu
from jax.experimental.pallas import tpu_sc as plsc

NX, NY = 432, 496
NUM_BEV = 64
VOXEL_X, VOXEL_Y = 0.16, 0.16
X_RANGE, Y_RANGE = 0.0, -39.68
P, B, M = 40000, 4, 40
PP = 40960
ROWS = PP // 128
NXP = NX
SP = NY * NXP
BSP = B * SP
NROWS = 898560
ZCH = 3744
BY = 16
JY = NY // BY


def _prep_body(bidx_ref, yc_ref, xc_ref, gt_ref, f_ref,
               dest_ref, code_ref, pf_ref, trig_ref):
    g = pl.program_id(0)

    @pl.when(g == 0)
    def _():
        h = gt_ref[6:7, :]
        trig_ref[0:1, :] = jnp.cos(h)
        trig_ref[1:2, :] = jnp.sin(h)

    pf_ref[:, 0:NUM_BEV] = f_ref[...]

    b = bidx_ref[...]
    y = yc_ref[...]
    x = xc_ref[...]
    px = x.astype(jnp.float32) * VOXEL_X + X_RANGE
    py = y.astype(jnp.float32) * VOXEL_Y + Y_RANGE
    cls = jnp.full((16, 128), -1, jnp.int32)
    for bb in range(B):
        bm = b == bb
        for m in range(M):
            k = bb * M + m
            cx = gt_ref[0, k]
            cy = gt_ref[1, k]
            dx2 = gt_ref[3, k] * 0.5
            dy2 = gt_ref[4, k] * 0.5
            ch = trig_ref[0, k]
            sh = trig_ref[1, k]
            ddx = px - cx
            ddy = py - cy
            lx = ddx * ch + ddy * sh
            ly = ddy * ch - ddx * sh
            hit = (jnp.abs(lx) <= dx2) & (jnp.abs(ly) <= dy2) & bm & (cls < 0)
            cls = jnp.where(hit, m, cls)
    r = lax.broadcasted_iota(jnp.int32, (16, 128), 0)
    c = lax.broadcasted_iota(jnp.int32, (16, 128), 1)
    p = (g * 16 + r) * 128 + c
    dest = b * SP + y * NXP + x
    dest_ref[...] = jnp.where(p < P, dest, BSP + p)
    code_ref[...] = p * 64 + (cls + 2)


_prep = pl.pallas_call(
    _prep_body,
    grid=(ROWS // 16,),
    in_specs=[
        pl.BlockSpec((16, 128), lambda g: (g, 0)),
        pl.BlockSpec((16, 128), lambda g: (g, 0)),
        pl.BlockSpec((16, 128), lambda g: (g, 0)),
        pl.BlockSpec((8, 256), lambda g: (0, 0)),
        pl.BlockSpec((2048, NUM_BEV), lambda g: (g, 0)),
    ],
    out_specs=[
        pl.BlockSpec((16, 128), lambda g: (g, 0)),
        pl.BlockSpec((16, 128), lambda g: (g, 0)),
        pl.BlockSpec((2048, 128), lambda g: (g, 0)),
    ],
    out_shape=[
        jax.ShapeDtypeStruct((ROWS, 128), jnp.int32),
        jax.ShapeDtypeStruct((ROWS, 128), jnp.int32),
        jax.ShapeDtypeStruct((PP, 128), jnp.float32),
    ],
    scratch_shapes=[pltpu.VMEM((8, 256), jnp.float32)],
)

_mesh = plsc.VectorSubcoreMesh(core_axis_name="c", subcore_axis_name="s")


@functools.partial(
    pl.kernel,
    mesh=_mesh,
    out_type=jax.ShapeDtypeStruct((NROWS,), jnp.int32),
    scratch_types=[
        pltpu.VMEM((ZCH,), jnp.int32),
        pltpu.VMEM((16, 128), jnp.int32),
        pltpu.VMEM((16, 128), jnp.int32),
        pltpu.SemaphoreType.DMA,
    ],
)
def _sc_map(dest_hbm, code_hbm, map_hbm, zbuf, d_v, c_v, sem):
    cid = lax.axis_index("c")
    sid = lax.axis_index("s")

    @pl.when(cid == 0)
    def _():
        @pl.loop(0, ZCH, step=16)
        def _(i):
            zbuf[pl.ds(i, 16)] = jnp.zeros((16,), jnp.int32)

        base = sid * (NROWS // 16)

        @pl.loop(0, NROWS // 16 // ZCH)
        def _(j):
            pltpu.sync_copy(zbuf, map_hbm.at[pl.ds(base + j * ZCH, ZCH)])

        plsc.subcore_barrier()

        @pl.when(sid == 0)
        def _():
            @pl.loop(0, ROWS // 16)
            def _(g):
                pltpu.sync_copy(dest_hbm.at[pl.ds(g * 16, 16)], d_v)
                pltpu.sync_copy(code_hbm.at[pl.ds(g * 16, 16)], c_v)
                for j in range(16):
                    pltpu.async_copy(c_v.at[j], map_hbm.at[d_v.at[j]], sem).wait()


@functools.partial(
    pl.kernel,
    mesh=_mesh,
    out_type=jax.ShapeDtypeStruct((NROWS, 128), jnp.float32),
    scratch_types=[
        pltpu.VMEM((128,), jnp.int32),
        pltpu.VMEM((128,), jnp.int32),
        pltpu.VMEM((128,), jnp.int32),
        pltpu.VMEM((128,), jnp.int32),
        pltpu.VMEM((128, 128), jnp.float32),
        pltpu.SemaphoreType.DMA,
    ],
)
def _sc_scatter(dest_hbm, code_hbm, map_hbm, feat_hbm, out_hbm,
                d_v, c_v, m_v, d2_v, f_v, sem):
    cid = lax.axis_index("c")
    sid = lax.axis_index("s")
    wid = sid * 2 + cid

    @pl.loop(0, ROWS // 32)
    def _(i):
        row = wid * (ROWS // 32) + i
        pltpu.sync_copy(dest_hbm.at[row], d_v)
        pltpu.sync_copy(code_hbm.at[row], c_v)
        pltpu.async_copy(map_hbm.at[d_v], m_v, sem).wait()
        pltpu.sync_copy(feat_hbm.at[pl.ds(row * 128, 128)], f_v)
        for k in range(8):
            sl = pl.ds(k * 16, 16)
            dd = d_v[sl]
            cc = c_v[sl]
            mm = m_v[sl]
            lane = lax.iota(jnp.int32, 16)
            dump = BSP + row * 128 + k * 16 + lane
            d2_v[sl] = jnp.where(mm == cc, dd, dump)
        pltpu.async_copy(f_v, out_hbm.at[d2_v], sem).wait()


def _fin_body(w_ref, b_ref, map_ref, feat_ref, sp_ref, off_ref, cls_ref):
    m2 = map_ref[...]
    valid2 = m2 > 0
    cls_ref[0] = jnp.where(valid2, (m2 & 63) - 2, -1)[:, :NX]
    for y in range(BY):
        t = feat_ref[pl.ds(y * NXP, NXP), 0:NUM_BEV].T
        v = valid2[y:y + 1, :]
        tm = jnp.where(v, t, 0.0)
        sp_ref[0, :, y, :] = tm[:, :NX]
        o = lax.dot_general(w_ref[...], tm, (((1,), (0,)), ((), ())),
                            preferred_element_type=jnp.float32)
        o = o + v.astype(jnp.float32) * b_ref[:, 0:1]
        off_ref[0, :, y, :] = o[0:2, :NX]


_fin = pl.pallas_call(
    _fin_body,
    grid=(B, JY),
    in_specs=[
        pl.BlockSpec((8, NUM_BEV), lambda b, j: (0, 0)),
        pl.BlockSpec((8, 8), lambda b, j: (0, 0)),
        pl.BlockSpec((BY, NXP), lambda b, j: (b * JY + j, 0)),
        pl.BlockSpec((BY * NXP, 128), lambda b, j: (b * JY + j, 0)),
    ],
    out_specs=[
        pl.BlockSpec((1, NUM_BEV, BY, NX), lambda b, j: (b, 0, j, 0)),
        pl.BlockSpec((1, 2, BY, NX), lambda b, j: (b, 0, j, 0)),
        pl.BlockSpec((1, BY, NX), lambda b, j: (b, j, 0)),
    ],
    out_shape=[
        jax.ShapeDtypeStruct((B, NUM_BEV, NY, NX), jnp.float32),
        jax.ShapeDtypeStruct((B, 2, NY, NX), jnp.float32),
        jax.ShapeDtypeStruct((B, NY, NX), jnp.int32),
    ],
)


def kernel(pillar_features, voxel_coords, gt_boxes, W_vote, b_vote):
    coords = jnp.pad(voxel_coords, ((0, PP - P), (0, 0)))
    bidx2 = coords[:, 0].reshape(ROWS, 128)
    yc2 = coords[:, 2].reshape(ROWS, 128)
    xc2 = coords[:, 3].reshape(ROWS, 128)
    gtt = jnp.pad(gt_boxes[..., :7].reshape(B * M, 7).T, ((0, 1), (0, 96)))
    w8 = jnp.zeros((8, NUM_BEV), jnp.float32).at[:2].set(W_vote)
    b8 = jnp.zeros((8, 8), jnp.float32).at[:2, 0].set(b_vote)

    dest2d, code2d, pf = _prep(bidx2, yc2, xc2, gtt, pillar_features)
    cmap = _sc_map(dest2d, code2d)
    feats = _sc_scatter(dest2d, code2d, cmap, pf)
    map2 = cmap.reshape(NROWS // NXP, NXP)
    spatial_features, offT, batch_cls = _fin(w8, b8, map2, feats)
    centering_offset = offT.transpose(0, 2, 3, 1)
    return spatial_features, centering_offset, batch_cls
